# baseline (speedup 1.0000x reference)
import functools

import jax
import jax.numpy as jnp
from jax import lax
from jax.experimental import pallas as pl
from jax.experimental.pallas import tpu as pltpu
from jax.experimental.pallas import tpu_sc as plsc


def kernel(x):
    M, N = x.shape
    info = plsc.get_sparse_core_info()
    NC, NS = info.num_cores, info.num_subcores
    NW = NC * NS
    rows_per_w = M // NW

    mesh = plsc.VectorSubcoreMesh(core_axis_name="c", subcore_axis_name="s")

    @functools.partial(
        pl.kernel,
        out_type=jax.ShapeDtypeStruct((M, N), x.dtype),
        mesh=mesh,
        scratch_types=[pltpu.SemaphoreType.DMA],
    )
    def copy_k(x_hbm, out_hbm, sem):
        wid = lax.axis_index("s") * NC + lax.axis_index("c")
        base = wid * rows_per_w
        src = x_hbm.at[pl.ds(base, rows_per_w), :]
        dst = out_hbm.at[pl.ds(base, rows_per_w), :]
        pltpu.async_copy(src, dst, sem).wait()

    gathered = copy_k(x)
    sizes = jnp.asarray([M], dtype=jnp.int32)
    return (gathered, sizes)


# speedup vs baseline: 1.0046x; 1.0046x over previous
import jax
import jax.numpy as jnp
from jax.experimental import pallas as pl
from jax.experimental.pallas import tpu as pltpu


def _dma_body(in_ref, out_ref, sem):
    copy = pltpu.make_async_copy(in_ref, out_ref, sem)
    copy.start()
    copy.wait()


def kernel(x):
    M, N = x.shape
    gathered = pl.pallas_call(
        _dma_body,
        in_specs=[pl.BlockSpec(memory_space=pltpu.MemorySpace.HBM)],
        out_specs=pl.BlockSpec(memory_space=pltpu.MemorySpace.HBM),
        out_shape=jax.ShapeDtypeStruct((M, N), x.dtype),
        scratch_shapes=[pltpu.SemaphoreType.DMA],
    )(x)
    sizes = jnp.asarray([M], dtype=jnp.int32)
    return (gathered, sizes)


# speedup vs baseline: 46.9747x; 46.7574x over previous
import jax
import jax.numpy as jnp
from jax.experimental import pallas as pl
from jax.experimental.pallas import tpu as pltpu

_CH = 256
_K = 8
_L = 4


def _make_body(M, N):
    nch = M // _CH

    def body(in_hbm, out_hbm, bufs, in_sems, out_sems):
        def in_copy(i):
            return pltpu.make_async_copy(
                in_hbm.at[pl.ds(i * _CH, _CH), :],
                bufs.at[i % _K],
                in_sems.at[i % _K],
            )

        def out_copy(i):
            return pltpu.make_async_copy(
                bufs.at[i % _K],
                out_hbm.at[pl.ds(i * _CH, _CH), :],
                out_sems.at[i % _K],
            )

        for i in range(-_L, nch):
            if i >= 0:
                in_copy(i).wait()
                out_copy(i).start()
            j = i + _L
            if 0 <= j < nch:
                if j >= _K:
                    out_copy(j - _K).wait()
                in_copy(j).start()
        for i in range(max(0, nch - _K), nch):
            out_copy(i).wait()

    return body


def kernel(x):
    M, N = x.shape
    gathered = pl.pallas_call(
        _make_body(M, N),
        in_specs=[pl.BlockSpec(memory_space=pltpu.MemorySpace.HBM)],
        out_specs=pl.BlockSpec(memory_space=pltpu.MemorySpace.HBM),
        out_shape=jax.ShapeDtypeStruct((M, N), x.dtype),
        scratch_shapes=[
            pltpu.VMEM((_K, _CH, N), x.dtype),
            pltpu.SemaphoreType.DMA((_K,)),
            pltpu.SemaphoreType.DMA((_K,)),
        ],
    )(x)
    sizes = jnp.asarray([M], dtype=jnp.int32)
    return (gathered, sizes)


# speedup vs baseline: 48.1639x; 1.0253x over previous
import jax
import jax.numpy as jnp
from jax.experimental import pallas as pl
from jax.experimental.pallas import tpu as pltpu

_CH = 512
_K = 8
_L = 4


def _make_body(M, N):
    nch = M // _CH

    def body(in_hbm, out_hbm, bufs, in_sems, out_sems):
        def in_copy(i):
            return pltpu.make_async_copy(
                in_hbm.at[pl.ds(i * _CH, _CH), :],
                bufs.at[i % _K],
                in_sems.at[i % _K],
            )

        def out_copy(i):
            return pltpu.make_async_copy(
                bufs.at[i % _K],
                out_hbm.at[pl.ds(i * _CH, _CH), :],
                out_sems.at[i % _K],
            )

        for i in range(-_L, nch):
            if i >= 0:
                in_copy(i).wait()
                out_copy(i).start()
            j = i + _L
            if 0 <= j < nch:
                if j >= _K:
                    out_copy(j - _K).wait()
                in_copy(j).start()
        for i in range(max(0, nch - _K), nch):
            out_copy(i).wait()

    return body


def kernel(x):
    M, N = x.shape
    gathered = pl.pallas_call(
        _make_body(M, N),
        in_specs=[pl.BlockSpec(memory_space=pltpu.MemorySpace.HBM)],
        out_specs=pl.BlockSpec(memory_space=pltpu.MemorySpace.HBM),
        out_shape=jax.ShapeDtypeStruct((M, N), x.dtype),
        scratch_shapes=[
            pltpu.VMEM((_K, _CH, N), x.dtype),
            pltpu.SemaphoreType.DMA((_K,)),
            pltpu.SemaphoreType.DMA((_K,)),
        ],
    )(x)
    sizes = jnp.asarray([M], dtype=jnp.int32)
    return (gathered, sizes)


# speedup vs baseline: 48.2417x; 1.0016x over previous
import jax
import jax.numpy as jnp
from jax.experimental import pallas as pl
from jax.experimental.pallas import tpu as pltpu

_CH = 512
_K = 16
_L = 8


def _make_body(M, N):
    nch = M // _CH

    def body(in_hbm, out_hbm, bufs, in_sems, out_sems):
        def in_copy(i):
            return pltpu.make_async_copy(
                in_hbm.at[pl.ds(i * _CH, _CH), :],
                bufs.at[i % _K],
                in_sems.at[i % _K],
            )

        def out_copy(i):
            return pltpu.make_async_copy(
                bufs.at[i % _K],
                out_hbm.at[pl.ds(i * _CH, _CH), :],
                out_sems.at[i % _K],
            )

        for i in range(-_L, nch):
            if i >= 0:
                in_copy(i).wait()
                out_copy(i).start()
            j = i + _L
            if 0 <= j < nch:
                if j >= _K:
                    out_copy(j - _K).wait()
                in_copy(j).start()
        for i in range(max(0, nch - _K), nch):
            out_copy(i).wait()

    return body


def kernel(x):
    M, N = x.shape
    gathered = pl.pallas_call(
        _make_body(M, N),
        in_specs=[pl.BlockSpec(memory_space=pltpu.MemorySpace.HBM)],
        out_specs=pl.BlockSpec(memory_space=pltpu.MemorySpace.HBM),
        out_shape=jax.ShapeDtypeStruct((M, N), x.dtype),
        scratch_shapes=[
            pltpu.VMEM((_K, _CH, N), x.dtype),
            pltpu.SemaphoreType.DMA((_K,)),
            pltpu.SemaphoreType.DMA((_K,)),
        ],
    )(x)
    sizes = jnp.asarray([M], dtype=jnp.int32)
    return (gathered, sizes)
